# baseline (speedup 1.0000x reference)
import jax
import jax.numpy as jnp
from jax.experimental import pallas as pl

B = 8192
D = 256
K = 1024
E = 8
BETA = 1.0

BM = 512
NB = B // BM

_HI = jax.lax.Precision.HIGHEST
_DEF = jax.lax.Precision.DEFAULT


def _stage(r, cb, rsumv, csumv):
    m = jax.lax.dot_general(r, cb, (((1,), (1,)), ((), ())), precision=_DEF)
    d = (rsumv - 2.0 * m) + csumv
    dmin = jnp.min(d, axis=1, keepdims=True)
    iota = jax.lax.broadcasted_iota(jnp.int32, (BM, K), 1)
    idx = jnp.min(jnp.where(d == dmin, iota, K), axis=1).astype(jnp.int32)
    onehot = (idx[:, None] ==
              jax.lax.broadcasted_iota(jnp.int32, (BM, K), 1)
              ).astype(jnp.float32)
    q = jax.lax.dot_general(onehot, cb, (((1,), (0,)), ((), ())),
                            precision=_HI)
    diff = q - r
    sse = jnp.sum(diff * diff)
    x_res = r + diff
    r_next = r - x_res
    return idx, x_res, r_next, sse


def _stage0_body(x_ref, cb_ref, rsum_ref, csum_ref, gw_ref, gb_ref,
                 rn_ref, xres_ref, idx_ref, ex_ref, sse_ref):
    x = x_ref[...]
    idx, x_res, r_next, sse = _stage(x, cb_ref[...], rsum_ref[...],
                                     csum_ref[...])
    logits = jax.lax.dot_general(x, gw_ref[...], (((1,), (0,)), ((), ())),
                                 precision=_DEF) + gb_ref[...]
    probs = jax.nn.softmax(logits, axis=-1)
    pmax = jnp.max(probs, axis=-1, keepdims=True)
    iota_e = jax.lax.broadcasted_iota(jnp.int32, (BM, E), 1)
    expert = jnp.min(jnp.where(probs == pmax, iota_e, E),
                     axis=1).astype(jnp.int32)
    rn_ref[...] = r_next
    xres_ref[...] = x_res
    idx_ref[...] = idx[:, None]
    ex_ref[...] = expert[:, None]
    sse_ref[...] = sse.reshape(1, 1, 1)


def _stage1_body(r_ref, cb_ref, rsum_ref, csum_ref,
                 rn_ref, xres_ref, idx_ref, sse_ref):
    idx, x_res, r_next, sse = _stage(r_ref[...], cb_ref[...], rsum_ref[...],
                                     csum_ref[...])
    rn_ref[...] = r_next
    xres_ref[...] = x_res
    idx_ref[...] = idx[:, None]
    sse_ref[...] = sse.reshape(1, 1, 1)


def _stage2_body(r_ref, cb_ref, rsum_ref, csum_ref, xres0_ref, xres1_ref,
                 xq_ref, idx_ref, sse_ref):
    idx, x_res, _, sse = _stage(r_ref[...], cb_ref[...], rsum_ref[...],
                                csum_ref[...])
    xq_ref[...] = (xres0_ref[...] + xres1_ref[...]) + x_res
    idx_ref[...] = idx[:, None]
    sse_ref[...] = sse.reshape(1, 1, 1)


_ROW = pl.BlockSpec((BM, D), lambda i: (i, 0))
_CB = pl.BlockSpec((K, D), lambda i: (0, 0))
_RS = pl.BlockSpec((BM, 1), lambda i: (i, 0))
_CS = pl.BlockSpec((1, K), lambda i: (0, 0))
_IDX = pl.BlockSpec((BM, 1), lambda i: (i, 0))
_SSE = pl.BlockSpec((1, 1, 1), lambda i: (i, 0, 0))

_ROW_SH = jax.ShapeDtypeStruct((B, D), jnp.float32)
_IDX_SH = jax.ShapeDtypeStruct((B, 1), jnp.int32)
_SSE_SH = jax.ShapeDtypeStruct((NB, 1, 1), jnp.float32)


def kernel(x, codebook_0, codebook_1, codebook_2, gate_W, gate_b,
           labels_0, labels_1, labels_2):
    del labels_0, labels_1, labels_2
    gate_b2 = gate_b.reshape(1, E)
    csums = [jnp.sum(cb ** 2, axis=1)[None, :]
             for cb in (codebook_0, codebook_1, codebook_2)]

    rsum0 = jnp.sum(x ** 2, axis=1, keepdims=True)
    r1, xres0, idx0, expert, sse0 = pl.pallas_call(
        _stage0_body,
        grid=(NB,),
        in_specs=[_ROW, _CB, _RS, _CS,
                  pl.BlockSpec((D, E), lambda i: (0, 0)),
                  pl.BlockSpec((1, E), lambda i: (0, 0))],
        out_specs=[_ROW, _ROW, _IDX, _IDX, _SSE],
        out_shape=[_ROW_SH, _ROW_SH, _IDX_SH, _IDX_SH, _SSE_SH],
    )(x, codebook_0, rsum0, csums[0], gate_W, gate_b2)

    rsum1 = jnp.sum(r1 ** 2, axis=1, keepdims=True)
    r2, xres1, idx1, sse1 = pl.pallas_call(
        _stage1_body,
        grid=(NB,),
        in_specs=[_ROW, _CB, _RS, _CS],
        out_specs=[_ROW, _ROW, _IDX, _SSE],
        out_shape=[_ROW_SH, _ROW_SH, _IDX_SH, _SSE_SH],
    )(r1, codebook_1, rsum1, csums[1])

    rsum2 = jnp.sum(r2 ** 2, axis=1, keepdims=True)
    xq, idx2, sse2 = pl.pallas_call(
        _stage2_body,
        grid=(NB,),
        in_specs=[_ROW, _CB, _RS, _CS, _ROW, _ROW],
        out_specs=[_ROW, _IDX, _SSE],
        out_shape=[_ROW_SH, _IDX_SH, _SSE_SH],
    )(r2, codebook_2, rsum2, csums[2], xres0, xres1)

    mean_losses = ((jnp.sum(sse0) + jnp.sum(sse1) + jnp.sum(sse2))
                   * ((1.0 + BETA) / (3.0 * B * D)))
    all_indices = jnp.concatenate([idx0, idx1, idx2, expert], axis=1)
    return (xq, mean_losses, all_indices)


# speedup vs baseline: 1.1025x; 1.1025x over previous
import jax
import jax.numpy as jnp
from jax.experimental import pallas as pl

B = 8192
D = 256
K = 1024
E = 8
BETA = 1.0

BM = 512
NB = B // BM

_HI = jax.lax.Precision.HIGHEST
_DEF = jax.lax.Precision.DEFAULT


def _rvq_body(x_ref, cb0_ref, cb1_ref, cb2_ref, gw_ref, gb_ref,
              xq_ref, idx_ref, loss_ref):
    x = x_ref[...]
    r = x
    xq = jnp.zeros_like(x)
    idx_cols = []
    losses = []
    for cb_ref in (cb0_ref, cb1_ref, cb2_ref):
        cb = cb_ref[...]
        csum = jnp.sum(cb * cb, axis=1)
        rsum = jnp.sum(r * r, axis=1)
        m = jax.lax.dot_general(r, cb, (((1,), (1,)), ((), ())),
                                precision=_DEF)
        d = (rsum[:, None] - 2.0 * m) + csum[None, :]
        dmin = jnp.min(d, axis=1, keepdims=True)
        iota = jax.lax.broadcasted_iota(jnp.int32, (BM, K), 1)
        idx = jnp.min(jnp.where(d == dmin, iota, K), axis=1).astype(jnp.int32)
        onehot = (idx[:, None] == iota).astype(jnp.float32)
        q = jax.lax.dot_general(onehot, cb, (((1,), (0,)), ((), ())),
                                precision=_HI)
        diff = q - r
        losses.append(jnp.sum(diff * diff))
        x_res = r + diff
        r = r - x_res
        xq = xq + x_res
        idx_cols.append(idx)
    logits = jax.lax.dot_general(x, gw_ref[...], (((1,), (0,)), ((), ())),
                                 precision=_DEF) + gb_ref[...]
    probs = jax.nn.softmax(logits, axis=-1)
    pmax = jnp.max(probs, axis=-1, keepdims=True)
    iota_e = jax.lax.broadcasted_iota(jnp.int32, (BM, E), 1)
    expert = jnp.min(jnp.where(probs == pmax, iota_e, E),
                     axis=1).astype(jnp.int32)
    idx_cols.append(expert)

    xq_ref[...] = xq
    idx_ref[...] = jnp.stack(idx_cols, axis=-1)
    loss_ref[...] = jnp.stack(losses).reshape(1, 1, 3)


def kernel(x, codebook_0, codebook_1, codebook_2, gate_W, gate_b,
           labels_0, labels_1, labels_2):
    del labels_0, labels_1, labels_2
    gate_b2 = gate_b.reshape(1, E)
    xq, idx, loss_parts = pl.pallas_call(
        _rvq_body,
        grid=(NB,),
        in_specs=[
            pl.BlockSpec((BM, D), lambda i: (i, 0)),
            pl.BlockSpec((K, D), lambda i: (0, 0)),
            pl.BlockSpec((K, D), lambda i: (0, 0)),
            pl.BlockSpec((K, D), lambda i: (0, 0)),
            pl.BlockSpec((D, E), lambda i: (0, 0)),
            pl.BlockSpec((1, E), lambda i: (0, 0)),
        ],
        out_specs=[
            pl.BlockSpec((BM, D), lambda i: (i, 0)),
            pl.BlockSpec((BM, 4), lambda i: (i, 0)),
            pl.BlockSpec((1, 1, 3), lambda i: (i, 0, 0)),
        ],
        out_shape=[
            jax.ShapeDtypeStruct((B, D), jnp.float32),
            jax.ShapeDtypeStruct((B, 4), jnp.int32),
            jax.ShapeDtypeStruct((NB, 1, 3), jnp.float32),
        ],
    )(x, codebook_0, codebook_1, codebook_2, gate_W, gate_b2)
    mean_losses = jnp.sum(loss_parts) * ((1.0 + BETA) / (3.0 * B * D))
    return (xq, mean_losses, idx)


# speedup vs baseline: 1.6070x; 1.4576x over previous
import jax
import jax.numpy as jnp
from jax.experimental import pallas as pl

B = 8192
D = 256
K = 1024
E = 8
BETA = 1.0

BM = 512
NB = B // BM

_HI = jax.lax.Precision.HIGHEST
_DEF = jax.lax.Precision.DEFAULT


def _rvq_body(x_ref, cb0_ref, cb1_ref, cb2_ref, gw_ref, gb_ref,
              xq_ref, idx_ref, loss_ref):
    x = x_ref[...]
    r = x
    xq = jnp.zeros_like(x)
    idx_cols = []
    losses = []
    for cb_ref in (cb0_ref, cb1_ref, cb2_ref):
        cb = cb_ref[...]
        csum = jnp.sum(cb * cb, axis=1)
        rsum = jnp.sum(r * r, axis=1)
        m = jax.lax.dot_general(r, cb, (((1,), (1,)), ((), ())),
                                precision=_DEF)
        d = (rsum[:, None] - 2.0 * m) + csum[None, :]
        dmin = jnp.min(d, axis=1, keepdims=True)
        iota = jax.lax.broadcasted_iota(jnp.int32, (BM, K), 1)
        idx = jnp.min(jnp.where(d == dmin, iota, K), axis=1).astype(jnp.int32)
        onehot = (idx[:, None] == iota).astype(jnp.bfloat16)
        hi = cb.astype(jnp.bfloat16)
        rem = cb - hi.astype(jnp.float32)
        mid = rem.astype(jnp.bfloat16)
        lo = (rem - mid.astype(jnp.float32)).astype(jnp.bfloat16)
        dims = (((1,), (0,)), ((), ()))
        q = ((jax.lax.dot_general(onehot, hi, dims,
                                  preferred_element_type=jnp.float32)
              + jax.lax.dot_general(onehot, mid, dims,
                                    preferred_element_type=jnp.float32))
             + jax.lax.dot_general(onehot, lo, dims,
                                   preferred_element_type=jnp.float32))
        diff = q - r
        losses.append(jnp.sum(diff * diff))
        x_res = r + diff
        r = r - x_res
        xq = xq + x_res
        idx_cols.append(idx)
    logits = jax.lax.dot_general(x, gw_ref[...], (((1,), (0,)), ((), ())),
                                 precision=_DEF) + gb_ref[...]
    probs = jax.nn.softmax(logits, axis=-1)
    pmax = jnp.max(probs, axis=-1, keepdims=True)
    iota_e = jax.lax.broadcasted_iota(jnp.int32, (BM, E), 1)
    expert = jnp.min(jnp.where(probs == pmax, iota_e, E),
                     axis=1).astype(jnp.int32)
    idx_cols.append(expert)

    xq_ref[...] = xq
    idx_ref[...] = jnp.stack(idx_cols, axis=-1)
    loss_ref[...] = jnp.stack(losses).reshape(1, 1, 3)


def kernel(x, codebook_0, codebook_1, codebook_2, gate_W, gate_b,
           labels_0, labels_1, labels_2):
    del labels_0, labels_1, labels_2
    gate_b2 = gate_b.reshape(1, E)
    xq, idx, loss_parts = pl.pallas_call(
        _rvq_body,
        grid=(NB,),
        in_specs=[
            pl.BlockSpec((BM, D), lambda i: (i, 0)),
            pl.BlockSpec((K, D), lambda i: (0, 0)),
            pl.BlockSpec((K, D), lambda i: (0, 0)),
            pl.BlockSpec((K, D), lambda i: (0, 0)),
            pl.BlockSpec((D, E), lambda i: (0, 0)),
            pl.BlockSpec((1, E), lambda i: (0, 0)),
        ],
        out_specs=[
            pl.BlockSpec((BM, D), lambda i: (i, 0)),
            pl.BlockSpec((BM, 4), lambda i: (i, 0)),
            pl.BlockSpec((1, 1, 3), lambda i: (i, 0, 0)),
        ],
        out_shape=[
            jax.ShapeDtypeStruct((B, D), jnp.float32),
            jax.ShapeDtypeStruct((B, 4), jnp.int32),
            jax.ShapeDtypeStruct((NB, 1, 3), jnp.float32),
        ],
    )(x, codebook_0, codebook_1, codebook_2, gate_W, gate_b2)
    mean_losses = jnp.sum(loss_parts) * ((1.0 + BETA) / (3.0 * B * D))
    return (xq, mean_losses, idx)


# speedup vs baseline: 1.7696x; 1.1012x over previous
import jax
import jax.numpy as jnp
from jax.experimental import pallas as pl

B = 8192
D = 256
K = 1024
E = 8
BETA = 1.0

BM = 1024
NB = B // BM

_HI = jax.lax.Precision.HIGHEST
_DEF = jax.lax.Precision.DEFAULT


def _rvq_body(x_ref, cb0_ref, cb1_ref, cb2_ref, gw_ref, gb_ref,
              xq_ref, idx_ref, loss_ref):
    x = x_ref[...]
    r = x
    xq = jnp.zeros_like(x)
    idx_cols = []
    losses = []
    for cb_ref in (cb0_ref, cb1_ref, cb2_ref):
        cb = cb_ref[...]
        csum = jnp.sum(cb * cb, axis=1)
        rsum = jnp.sum(r * r, axis=1)
        m = jax.lax.dot_general(r, cb, (((1,), (1,)), ((), ())),
                                precision=_DEF)
        d = (rsum[:, None] - 2.0 * m) + csum[None, :]
        dmin = jnp.min(d, axis=1, keepdims=True)
        iota = jax.lax.broadcasted_iota(jnp.int32, (BM, K), 1)
        idx = jnp.min(jnp.where(d == dmin, iota, K), axis=1).astype(jnp.int32)
        onehot = (idx[:, None] == iota).astype(jnp.bfloat16)
        hi = cb.astype(jnp.bfloat16)
        rem = cb - hi.astype(jnp.float32)
        mid = rem.astype(jnp.bfloat16)
        lo = (rem - mid.astype(jnp.float32)).astype(jnp.bfloat16)
        dims = (((1,), (0,)), ((), ()))
        q = ((jax.lax.dot_general(onehot, hi, dims,
                                  preferred_element_type=jnp.float32)
              + jax.lax.dot_general(onehot, mid, dims,
                                    preferred_element_type=jnp.float32))
             + jax.lax.dot_general(onehot, lo, dims,
                                   preferred_element_type=jnp.float32))
        diff = q - r
        losses.append(jnp.sum(diff * diff))
        x_res = r + diff
        r = r - x_res
        xq = xq + x_res
        idx_cols.append(idx)
    logits = jax.lax.dot_general(x, gw_ref[...], (((1,), (0,)), ((), ())),
                                 precision=_DEF) + gb_ref[...]
    probs = jax.nn.softmax(logits, axis=-1)
    pmax = jnp.max(probs, axis=-1, keepdims=True)
    iota_e = jax.lax.broadcasted_iota(jnp.int32, (BM, E), 1)
    expert = jnp.min(jnp.where(probs == pmax, iota_e, E),
                     axis=1).astype(jnp.int32)
    idx_cols.append(expert)

    xq_ref[...] = xq
    idx_ref[...] = jnp.stack(idx_cols, axis=-1)
    loss_ref[...] = jnp.stack(losses).reshape(1, 1, 3)


def kernel(x, codebook_0, codebook_1, codebook_2, gate_W, gate_b,
           labels_0, labels_1, labels_2):
    del labels_0, labels_1, labels_2
    gate_b2 = gate_b.reshape(1, E)
    xq, idx, loss_parts = pl.pallas_call(
        _rvq_body,
        grid=(NB,),
        in_specs=[
            pl.BlockSpec((BM, D), lambda i: (i, 0)),
            pl.BlockSpec((K, D), lambda i: (0, 0)),
            pl.BlockSpec((K, D), lambda i: (0, 0)),
            pl.BlockSpec((K, D), lambda i: (0, 0)),
            pl.BlockSpec((D, E), lambda i: (0, 0)),
            pl.BlockSpec((1, E), lambda i: (0, 0)),
        ],
        out_specs=[
            pl.BlockSpec((BM, D), lambda i: (i, 0)),
            pl.BlockSpec((BM, 4), lambda i: (i, 0)),
            pl.BlockSpec((1, 1, 3), lambda i: (i, 0, 0)),
        ],
        out_shape=[
            jax.ShapeDtypeStruct((B, D), jnp.float32),
            jax.ShapeDtypeStruct((B, 4), jnp.int32),
            jax.ShapeDtypeStruct((NB, 1, 3), jnp.float32),
        ],
    )(x, codebook_0, codebook_1, codebook_2, gate_W, gate_b2)
    mean_losses = jnp.sum(loss_parts) * ((1.0 + BETA) / (3.0 * B * D))
    return (xq, mean_losses, idx)


# speedup vs baseline: 1.8319x; 1.0352x over previous
import jax
import jax.numpy as jnp
from jax.experimental import pallas as pl

B = 8192
D = 256
K = 1024
E = 8
BETA = 1.0

BM = 2048
NB = B // BM

_HI = jax.lax.Precision.HIGHEST
_DEF = jax.lax.Precision.DEFAULT


def _rvq_body(x_ref, cb0_ref, cb1_ref, cb2_ref, gw_ref, gb_ref,
              xq_ref, idx_ref, loss_ref):
    x = x_ref[...]
    r = x
    xq = jnp.zeros_like(x)
    idx_cols = []
    losses = []
    for cb_ref in (cb0_ref, cb1_ref, cb2_ref):
        cb = cb_ref[...]
        csum = jnp.sum(cb * cb, axis=1)
        rsum = jnp.sum(r * r, axis=1)
        m = jax.lax.dot_general(r, cb, (((1,), (1,)), ((), ())),
                                precision=_DEF)
        d = (rsum[:, None] - 2.0 * m) + csum[None, :]
        dmin = jnp.min(d, axis=1, keepdims=True)
        iota = jax.lax.broadcasted_iota(jnp.int32, (BM, K), 1)
        idx = jnp.min(jnp.where(d == dmin, iota, K), axis=1).astype(jnp.int32)
        onehot = (idx[:, None] == iota).astype(jnp.bfloat16)
        hi = cb.astype(jnp.bfloat16)
        rem = cb - hi.astype(jnp.float32)
        mid = rem.astype(jnp.bfloat16)
        lo = (rem - mid.astype(jnp.float32)).astype(jnp.bfloat16)
        dims = (((1,), (0,)), ((), ()))
        q = ((jax.lax.dot_general(onehot, hi, dims,
                                  preferred_element_type=jnp.float32)
              + jax.lax.dot_general(onehot, mid, dims,
                                    preferred_element_type=jnp.float32))
             + jax.lax.dot_general(onehot, lo, dims,
                                   preferred_element_type=jnp.float32))
        diff = q - r
        losses.append(jnp.sum(diff * diff))
        x_res = r + diff
        r = r - x_res
        xq = xq + x_res
        idx_cols.append(idx)
    logits = jax.lax.dot_general(x, gw_ref[...], (((1,), (0,)), ((), ())),
                                 precision=_DEF) + gb_ref[...]
    probs = jax.nn.softmax(logits, axis=-1)
    pmax = jnp.max(probs, axis=-1, keepdims=True)
    iota_e = jax.lax.broadcasted_iota(jnp.int32, (BM, E), 1)
    expert = jnp.min(jnp.where(probs == pmax, iota_e, E),
                     axis=1).astype(jnp.int32)
    idx_cols.append(expert)

    xq_ref[...] = xq
    idx_ref[...] = jnp.stack(idx_cols, axis=-1)
    loss_ref[...] = jnp.stack(losses).reshape(1, 1, 3)


def kernel(x, codebook_0, codebook_1, codebook_2, gate_W, gate_b,
           labels_0, labels_1, labels_2):
    del labels_0, labels_1, labels_2
    gate_b2 = gate_b.reshape(1, E)
    xq, idx, loss_parts = pl.pallas_call(
        _rvq_body,
        grid=(NB,),
        in_specs=[
            pl.BlockSpec((BM, D), lambda i: (i, 0)),
            pl.BlockSpec((K, D), lambda i: (0, 0)),
            pl.BlockSpec((K, D), lambda i: (0, 0)),
            pl.BlockSpec((K, D), lambda i: (0, 0)),
            pl.BlockSpec((D, E), lambda i: (0, 0)),
            pl.BlockSpec((1, E), lambda i: (0, 0)),
        ],
        out_specs=[
            pl.BlockSpec((BM, D), lambda i: (i, 0)),
            pl.BlockSpec((BM, 4), lambda i: (i, 0)),
            pl.BlockSpec((1, 1, 3), lambda i: (i, 0, 0)),
        ],
        out_shape=[
            jax.ShapeDtypeStruct((B, D), jnp.float32),
            jax.ShapeDtypeStruct((B, 4), jnp.int32),
            jax.ShapeDtypeStruct((NB, 1, 3), jnp.float32),
        ],
    )(x, codebook_0, codebook_1, codebook_2, gate_W, gate_b2)
    mean_losses = jnp.sum(loss_parts) * ((1.0 + BETA) / (3.0 * B * D))
    return (xq, mean_losses, idx)
